# Initial kernel scaffold; baseline (speedup 1.0000x reference)
import jax
import jax.numpy as jnp
from jax.experimental import pallas as pl


def kernel(node2node_shortest_path_length_type, codebook):
    raise NotImplementedError("write your pallas kernel here")


# speedup vs baseline: 24.8563x; 24.8563x over previous
import functools

import jax
import jax.numpy as jnp
from jax import lax
from jax.experimental import pallas as pl
from jax.experimental.pallas import tpu as pltpu
from jax.experimental.pallas import tpu_sc as plsc

NC = 2
NS = 16
NW = NC * NS
LANES = 16

C_BINS = 272


def _build_hist(n_rows: int, n_idx: int, rows_per_chunk: int):
    assert n_rows % NW == 0
    rows_per_w = n_rows // NW
    assert rows_per_w % rows_per_chunk == 0
    n_chunks = rows_per_w // rows_per_chunk
    ch_i = rows_per_chunk * n_idx
    ch_c = rows_per_chunk * C_BINS
    mesh = plsc.VectorSubcoreMesh(core_axis_name="c", subcore_axis_name="s")

    @functools.partial(
        pl.kernel,
        out_type=jax.ShapeDtypeStruct((n_rows * C_BINS,), jnp.float32),
        mesh=mesh,
        compiler_params=pltpu.CompilerParams(needs_layout_passes=False),
        scratch_types=[
            pltpu.VMEM((ch_i,), jnp.int32),
            pltpu.VMEM((ch_c,), jnp.float32),
        ],
    )
    def hist(idx_hbm, cnt_hbm, idx_v, cnt_v):
        wid = lax.axis_index("s") * NC + lax.axis_index("c")
        base = wid * rows_per_w
        zeros = jnp.zeros((LANES,), jnp.float32)
        ones = jnp.ones((LANES,), jnp.float32)
        neg_ones = -ones

        def zero_body(k, _):
            cnt_v[pl.ds(k * LANES, LANES)] = zeros
            return ()

        lax.fori_loop(0, ch_c // LANES, zero_body, ())

        def scatter_row(r, val):
            off = (r * C_BINS + jnp.zeros((LANES,), jnp.int32)).astype(jnp.int32)
            i0 = idx_v[pl.ds(r * n_idx, LANES)] + off
            i1 = idx_v[pl.ds(r * n_idx + LANES, LANES)] + off
            plsc.addupdate_scatter(cnt_v, [i0], val)
            plsc.addupdate_scatter(cnt_v, [i1], val)

        def add_body(r, _):
            scatter_row(r, ones)
            return ()

        def sub_body(r, _):
            scatter_row(r, neg_ones)
            return ()

        def chunk_body(ci, _):
            row0 = base + ci * rows_per_chunk
            pltpu.sync_copy(idx_hbm.at[pl.ds(row0 * n_idx, ch_i)], idx_v)
            lax.fori_loop(0, rows_per_chunk, add_body, ())
            pltpu.sync_copy(cnt_v, cnt_hbm.at[pl.ds(row0 * C_BINS, ch_c)])
            lax.fori_loop(0, rows_per_chunk, sub_body, ())
            return ()

        lax.fori_loop(0, n_chunks, chunk_body, ())

    return hist


def _mm_body(cnt_ref, cb_ref, o_ref):
    o_ref[...] = jnp.dot(
        cnt_ref[...], cb_ref[...], preferred_element_type=jnp.float32
    )


def _build_matmul(n_rows: int, d: int, block_rows: int):
    grid = (n_rows // block_rows,)
    return pl.pallas_call(
        _mm_body,
        grid=grid,
        in_specs=[
            pl.BlockSpec((block_rows, C_BINS), lambda i: (i, 0)),
            pl.BlockSpec((C_BINS, d), lambda i: (0, 0)),
        ],
        out_specs=pl.BlockSpec((block_rows, d), lambda i: (i, 0)),
        out_shape=jax.ShapeDtypeStruct((n_rows, d), jnp.float32),
    )


@functools.lru_cache(maxsize=None)
def _build(b, n, j, v, d):
    n_rows = b * n
    hist = _build_hist(n_rows, j, rows_per_chunk=128)
    matmul = _build_matmul(n_rows, d, block_rows=2048)

    def run(idx, codebook):
        idx_flat = idx.astype(jnp.int32).reshape(n_rows * j)
        counts = hist(idx_flat).reshape(n_rows, C_BINS)
        cb_pad = jnp.pad(codebook.astype(jnp.float32), ((0, C_BINS - v), (0, 0)))
        return matmul(counts, cb_pad).reshape(b, n, d)

    return run


def kernel(node2node_shortest_path_length_type, codebook):
    b, n, j = node2node_shortest_path_length_type.shape
    v, d = codebook.shape
    return _build(b, n, j, v, d)(node2node_shortest_path_length_type, codebook)


# speedup vs baseline: 36.1294x; 1.4535x over previous
import functools

import jax
import jax.numpy as jnp
from jax import lax
from jax.experimental import pallas as pl
from jax.experimental.pallas import tpu as pltpu
from jax.experimental.pallas import tpu_sc as plsc

NC = 2
NS = 16
NW = NC * NS
LANES = 16

C_BINS = 272


def _build_hist(b: int, n: int, n_idx: int, blocks_per_chunk: int):
    n_rows = b * n
    assert n_rows % NW == 0
    rows_per_w = n_rows // NW
    rows_per_chunk = blocks_per_chunk * n
    assert rows_per_w % rows_per_chunk == 0
    n_chunks = rows_per_w // rows_per_chunk
    blocks_per_w = rows_per_w // n
    mesh = plsc.VectorSubcoreMesh(core_axis_name="c", subcore_axis_name="s")

    @functools.partial(
        pl.kernel,
        out_type=jax.ShapeDtypeStruct((n_rows, C_BINS), jnp.float32),
        mesh=mesh,
        compiler_params=pltpu.CompilerParams(needs_layout_passes=False),
        scratch_types=[
            pltpu.VMEM((blocks_per_chunk, n, n_idx), jnp.int32),
            pltpu.VMEM((rows_per_chunk, C_BINS), jnp.float32),
        ],
    )
    def hist(idx_hbm, cnt_hbm, idx_v, cnt_v):
        wid = lax.axis_index("s") * NC + lax.axis_index("c")
        base_blk = wid * blocks_per_w
        zeros = jnp.zeros((LANES,), jnp.float32)
        ones = jnp.ones((LANES,), jnp.float32)
        neg_ones = -ones

        def zero_body(k, _):
            r = k // (C_BINS // LANES)
            c = k % (C_BINS // LANES)
            cnt_v[r, pl.ds(c * LANES, LANES)] = zeros
            return ()

        lax.fori_loop(0, rows_per_chunk * (C_BINS // LANES), zero_body, ())

        def scatter_row(r, val):
            rr = jnp.full((LANES,), r, jnp.int32)
            i0 = idx_v[r // n, r % n, pl.ds(0, LANES)]
            i1 = idx_v[r // n, r % n, pl.ds(LANES, LANES)]
            plsc.addupdate_scatter(cnt_v, [rr, i0], val)
            plsc.addupdate_scatter(cnt_v, [rr, i1], val)

        def add_body(r, _):
            scatter_row(r, ones)
            return ()

        def sub_body(r, _):
            scatter_row(r, neg_ones)
            return ()

        def chunk_body(ci, _):
            blk0 = base_blk + ci * blocks_per_chunk
            pltpu.sync_copy(idx_hbm.at[pl.ds(blk0, blocks_per_chunk)], idx_v)
            lax.fori_loop(0, rows_per_chunk, add_body, ())
            pltpu.sync_copy(cnt_v, cnt_hbm.at[pl.ds(blk0 * n, rows_per_chunk)])
            lax.fori_loop(0, rows_per_chunk, sub_body, ())
            return ()

        lax.fori_loop(0, n_chunks, chunk_body, ())

    return hist


def _mm_body(cnt_ref, cb_ref, o_ref):
    o_ref[...] = jnp.dot(
        cnt_ref[...], cb_ref[...], preferred_element_type=jnp.float32
    )


def _build_matmul(n_rows: int, d: int, block_rows: int):
    grid = (n_rows // block_rows,)
    return pl.pallas_call(
        _mm_body,
        grid=grid,
        in_specs=[
            pl.BlockSpec((block_rows, C_BINS), lambda i: (i, 0)),
            pl.BlockSpec((C_BINS, d), lambda i: (0, 0)),
        ],
        out_specs=pl.BlockSpec((block_rows, d), lambda i: (i, 0)),
        out_shape=jax.ShapeDtypeStruct((n_rows, d), jnp.float32),
    )


@functools.lru_cache(maxsize=None)
def _build(b, n, j, v, d):
    n_rows = b * n
    hist = _build_hist(b, n, j, blocks_per_chunk=4)
    matmul = _build_matmul(n_rows, d, block_rows=2048)

    def run(idx, codebook):
        counts = hist(idx.astype(jnp.int32))
        cb_pad = jnp.pad(codebook.astype(jnp.float32), ((0, C_BINS - v), (0, 0)))
        return matmul(counts, cb_pad).reshape(b, n, d)

    return run


def kernel(node2node_shortest_path_length_type, codebook):
    b, n, j = node2node_shortest_path_length_type.shape
    v, d = codebook.shape
    return _build(b, n, j, v, d)(node2node_shortest_path_length_type, codebook)
